# baseline (speedup 1.0000x reference)
import functools

import jax
import jax.numpy as jnp
from jax import lax
from jax.experimental import pallas as pl
from jax.experimental.pallas import tpu as pltpu
from jax.experimental.pallas import tpu_sc as plsc

_N = 10000
_D = 128
_E = 320000
_NC = 2
_NS = 16
_NW = _NC * _NS
_EPW = _E // _NW
_C = 40
_NCHUNK = _EPW // _C
_NB = 5
_NI = 2 * _NB
_RPT = 624
_REM = _N - _RPT * _NS


def _seg_sum_sc(h, src, dst):
    mesh = plsc.VectorSubcoreMesh(core_axis_name="c", subcore_axis_name="s",
                                  num_cores=_NC, num_subcores=_NS)

    @functools.partial(
        pl.kernel,
        out_type=jax.ShapeDtypeStruct((_NC, _N, _D), jnp.float32),
        mesh=mesh,
        scratch_types=[
            pltpu.VMEM((_NB, _C), jnp.int32),
            pltpu.VMEM((_NB, _C), jnp.int32),
            pltpu.VMEM((_NB, _C, _D), jnp.float32),
            pltpu.VMEM_SHARED((_N, _D), jnp.float32),
            pltpu.SemaphoreType.DMA((_NB,)),
            pltpu.SemaphoreType.DMA((_NB,)),
        ],
    )
    def body(h_hbm, src_hbm, dst_hbm, out_hbm, idx_s, idx_d, rows, agg,
             sem_i, sem_g):
        c = lax.axis_index("c")
        s = lax.axis_index("s")
        wid = s * _NC + c
        base = wid * _EPW

        for b in range(_NB):
            off = base + b * _C
            pltpu.async_copy(src_hbm.at[pl.ds(off, _C)], idx_s.at[b],
                             sem_i.at[b])
            pltpu.async_copy(dst_hbm.at[pl.ds(off, _C)], idx_d.at[b],
                             sem_i.at[b])
        zb = rows.at[_NB - 1]

        def zstore(i, carry):
            r = i // (_D // 16)
            col = (i % (_D // 16)) * 16
            zb[r, pl.ds(col, 16)] = jnp.zeros((16,), jnp.float32)
            return carry

        lax.fori_loop(0, _C * (_D // 16), zstore, 0)
        for b in range(_NB - 1):
            off = base + b * _C
            pltpu.make_async_copy(src_hbm.at[pl.ds(off, _C)],
                                  idx_s.at[b], sem_i.at[b]).wait()
            pltpu.make_async_copy(dst_hbm.at[pl.ds(off, _C)],
                                  idx_d.at[b], sem_i.at[b]).wait()
            pltpu.async_copy(h_hbm.at[idx_s.at[b]], rows.at[b], sem_g.at[b])
        row0 = s * _RPT
        for k in range(_RPT // _C):
            pltpu.sync_copy(zb, agg.at[pl.ds(row0 + k * _C, _C)])
        pltpu.sync_copy(zb.at[pl.ds(0, _RPT % _C)],
                        agg.at[pl.ds(row0 + (_RPT // _C) * _C, _RPT % _C)])

        @pl.when(s == 0)
        def _():
            pltpu.sync_copy(zb.at[pl.ds(0, _REM)],
                            agg.at[pl.ds(_RPT * _NS, _REM)])

        plsc.subcore_barrier()

        def group(g, carry):
            for b in range(_NB):
                i = g * _NB + b
                ci, cg, cs = i, i - 1, i - 3

                @pl.when(jnp.logical_and(ci >= _NB, ci < _NCHUNK))
                def _():
                    off = base + ci * _C
                    pltpu.async_copy(src_hbm.at[pl.ds(off, _C)],
                                     idx_s.at[b], sem_i.at[b])
                    pltpu.async_copy(dst_hbm.at[pl.ds(off, _C)],
                                     idx_d.at[b], sem_i.at[b])

                bg = (b - 1) % _NB

                @pl.when(jnp.logical_and(cg >= _NB - 1, cg < _NCHUNK))
                def _():
                    off = base + cg * _C
                    pltpu.make_async_copy(src_hbm.at[pl.ds(off, _C)],
                                          idx_s.at[bg], sem_i.at[bg]).wait()
                    pltpu.make_async_copy(dst_hbm.at[pl.ds(off, _C)],
                                          idx_d.at[bg], sem_i.at[bg]).wait()
                    pltpu.async_copy(h_hbm.at[idx_s.at[bg]], rows.at[bg],
                                     sem_g.at[bg])

                bs = (b - 3) % _NB

                @pl.when(jnp.logical_and(cs >= 0, cs < _NCHUNK))
                def _():
                    pltpu.make_async_copy(h_hbm.at[idx_s.at[bs]],
                                          rows.at[bs], sem_g.at[bs]).wait()
                    pltpu.sync_copy(rows.at[bs], agg.at[idx_d.at[bs]],
                                    add=True)
            return carry

        lax.fori_loop(0, (_NCHUNK + _NB - 1) // _NB + 1, group, 0)

        plsc.subcore_barrier()
        pltpu.sync_copy(agg.at[pl.ds(row0, _RPT)],
                        out_hbm.at[c, pl.ds(row0, _RPT)])

        @pl.when(s == 0)
        def _():
            pltpu.sync_copy(agg.at[pl.ds(_RPT * _NS, _REM)],
                            out_hbm.at[c, pl.ds(_RPT * _NS, _REM)])

    return body(h, src, dst)


def _gin_mlp(h_ref, p_ref, eps_ref, w1_ref, b1_ref, g_ref, be_ref, w2_ref, b2_ref):
    eps = eps_ref[0]
    comb = (1.0 + eps) * h_ref[...] + p_ref[0] + p_ref[1]
    y = jnp.dot(comb, w1_ref[...], preferred_element_type=jnp.float32) + b1_ref[...]
    mu = jnp.mean(y, axis=0, keepdims=True)
    var = jnp.mean((y - mu) ** 2, axis=0, keepdims=True)
    yn = (y - mu) * lax.rsqrt(var + 1e-5) * g_ref[...] + be_ref[...]
    z = jnp.maximum(yn, 0.0)
    h2 = jnp.dot(z, w2_ref[...], preferred_element_type=jnp.float32) + b2_ref[...]
    return jnp.maximum(h2, 0.0)


def _tc_layer(h, p, eps, w1, b1, g, be, w2, b2):
    def kfn(h_ref, p_ref, eps_ref, w1_ref, b1_ref, g_ref, be_ref, w2_ref,
            b2_ref, o_ref):
        o_ref[...] = _gin_mlp(h_ref, p_ref, eps_ref, w1_ref, b1_ref, g_ref,
                              be_ref, w2_ref, b2_ref)

    specs = [pl.BlockSpec(memory_space=pltpu.VMEM) for _ in range(9)]
    specs[2] = pl.BlockSpec(memory_space=pltpu.SMEM)
    return pl.pallas_call(
        kfn,
        out_shape=jax.ShapeDtypeStruct((_N, _D), jnp.float32),
        in_specs=specs,
        out_specs=pl.BlockSpec(memory_space=pltpu.VMEM),
    )(h, p, eps.reshape(1), w1, b1.reshape(1, -1), g.reshape(1, -1),
      be.reshape(1, -1), w2, b2.reshape(1, -1))


def _tc_layer_final(h, p, eps, w1, b1, g, be, w2, b2, wf1, bf1, wf2, bf2):
    def kfn(h_ref, p_ref, eps_ref, w1_ref, b1_ref, g_ref, be_ref, w2_ref,
            b2_ref, wf1_ref, bf1_ref, wf2_ref, bf2_ref, o_ref):
        h2 = _gin_mlp(h_ref, p_ref, eps_ref, w1_ref, b1_ref, g_ref, be_ref,
                      w2_ref, b2_ref)
        t = jnp.maximum(
            jnp.dot(h2, wf1_ref[...], preferred_element_type=jnp.float32)
            + bf1_ref[...], 0.0)
        o_ref[...] = (jnp.dot(t, wf2_ref[...], preferred_element_type=jnp.float32)
                      + bf2_ref[...])

    specs = [pl.BlockSpec(memory_space=pltpu.VMEM) for _ in range(13)]
    specs[2] = pl.BlockSpec(memory_space=pltpu.SMEM)
    return pl.pallas_call(
        kfn,
        out_shape=jax.ShapeDtypeStruct((_N, _D), jnp.float32),
        in_specs=specs,
        out_specs=pl.BlockSpec(memory_space=pltpu.VMEM),
    )(h, p, eps.reshape(1), w1, b1.reshape(1, -1), g.reshape(1, -1),
      be.reshape(1, -1), w2, b2.reshape(1, -1), wf1, bf1.reshape(1, -1),
      wf2, bf2.reshape(1, -1))


def kernel(x, edge_index,
           eps0, W1_0, b1_0, g_0, be_0, W2_0, b2_0,
           eps1, W1_1, b1_1, g_1, be_1, W2_1, b2_1,
           eps2, W1_2, b1_2, g_2, be_2, W2_2, b2_2,
           Wf1, bf1, Wf2, bf2):
    src = edge_index[0]
    dst = edge_index[1]
    layers = [
        (eps0, W1_0, b1_0, g_0, be_0, W2_0, b2_0),
        (eps1, W1_1, b1_1, g_1, be_1, W2_1, b2_1),
        (eps2, W1_2, b1_2, g_2, be_2, W2_2, b2_2),
    ]
    h = x
    for l, (eps, w1, b1, g, be, w2, b2) in enumerate(layers):
        p = _seg_sum_sc(h, src, dst)
        if l < 2:
            h = _tc_layer(h, p, eps, w1, b1, g, be, w2, b2)
        else:
            h = _tc_layer_final(h, p, eps, w1, b1, g, be, w2, b2,
                                Wf1, bf1, Wf2, bf2)
    return h


# speedup vs baseline: 1.1904x; 1.1904x over previous
import functools

import jax
import jax.numpy as jnp
from jax import lax
from jax.experimental import pallas as pl
from jax.experimental.pallas import tpu as pltpu
from jax.experimental.pallas import tpu_sc as plsc

_N = 10000
_D = 128
_E = 320000
_NC = 2
_NS = 16
_NW = _NC * _NS
_EPW = _E // _NW
_C = 80
_NCHUNK = _EPW // _C
_NB = 4
_NI = 2 * _NB
_RPT = 624
_REM = _N - _RPT * _NS


def _seg_sum_sc(h, src, dst):
    mesh = plsc.VectorSubcoreMesh(core_axis_name="c", subcore_axis_name="s",
                                  num_cores=_NC, num_subcores=_NS)

    @functools.partial(
        pl.kernel,
        out_type=jax.ShapeDtypeStruct((_NC, _N, _D), jnp.float32),
        mesh=mesh,
        scratch_types=[
            pltpu.VMEM((_NB, _C), jnp.int32),
            pltpu.VMEM((_NB, _C), jnp.int32),
            pltpu.VMEM((_NB, _C, _D), jnp.float32),
            pltpu.VMEM_SHARED((_N, _D), jnp.float32),
            pltpu.SemaphoreType.DMA((_NB,)),
            pltpu.SemaphoreType.DMA((_NB,)),
        ],
    )
    def body(h_hbm, src_hbm, dst_hbm, out_hbm, idx_s, idx_d, rows, agg,
             sem_i, sem_g):
        c = lax.axis_index("c")
        s = lax.axis_index("s")
        wid = s * _NC + c
        base = wid * _EPW

        for b in range(_NB):
            off = base + b * _C
            pltpu.async_copy(src_hbm.at[pl.ds(off, _C)], idx_s.at[b],
                             sem_i.at[b])
            pltpu.async_copy(dst_hbm.at[pl.ds(off, _C)], idx_d.at[b],
                             sem_i.at[b])
        zb = rows.at[_NB - 1]

        def zstore(i, carry):
            r = i // (_D // 16)
            col = (i % (_D // 16)) * 16
            zb[r, pl.ds(col, 16)] = jnp.zeros((16,), jnp.float32)
            return carry

        lax.fori_loop(0, _C * (_D // 16), zstore, 0)
        for b in range(_NB - 1):
            off = base + b * _C
            pltpu.make_async_copy(src_hbm.at[pl.ds(off, _C)],
                                  idx_s.at[b], sem_i.at[b]).wait()
            pltpu.make_async_copy(dst_hbm.at[pl.ds(off, _C)],
                                  idx_d.at[b], sem_i.at[b]).wait()
            pltpu.async_copy(h_hbm.at[idx_s.at[b]], rows.at[b], sem_g.at[b])
        row0 = s * _RPT
        for k in range(_RPT // _C):
            pltpu.sync_copy(zb, agg.at[pl.ds(row0 + k * _C, _C)])
        pltpu.sync_copy(zb.at[pl.ds(0, _RPT % _C)],
                        agg.at[pl.ds(row0 + (_RPT // _C) * _C, _RPT % _C)])

        @pl.when(s == 0)
        def _():
            pltpu.sync_copy(zb.at[pl.ds(0, _REM)],
                            agg.at[pl.ds(_RPT * _NS, _REM)])

        plsc.subcore_barrier()

        def group(g, carry):
            for b in range(_NB):
                i = g * _NB + b
                ci, cg, cs = i, i - 1, i - 3

                @pl.when(jnp.logical_and(ci >= _NB, ci < _NCHUNK))
                def _():
                    off = base + ci * _C
                    pltpu.async_copy(src_hbm.at[pl.ds(off, _C)],
                                     idx_s.at[b], sem_i.at[b])
                    pltpu.async_copy(dst_hbm.at[pl.ds(off, _C)],
                                     idx_d.at[b], sem_i.at[b])

                bg = (b - 1) % _NB

                @pl.when(jnp.logical_and(cg >= _NB - 1, cg < _NCHUNK))
                def _():
                    off = base + cg * _C
                    pltpu.make_async_copy(src_hbm.at[pl.ds(off, _C)],
                                          idx_s.at[bg], sem_i.at[bg]).wait()
                    pltpu.make_async_copy(dst_hbm.at[pl.ds(off, _C)],
                                          idx_d.at[bg], sem_i.at[bg]).wait()
                    pltpu.async_copy(h_hbm.at[idx_s.at[bg]], rows.at[bg],
                                     sem_g.at[bg])

                bs = (b - 3) % _NB

                @pl.when(jnp.logical_and(cs >= 0, cs < _NCHUNK))
                def _():
                    pltpu.make_async_copy(h_hbm.at[idx_s.at[bs]],
                                          rows.at[bs], sem_g.at[bs]).wait()
                    pltpu.sync_copy(rows.at[bs], agg.at[idx_d.at[bs]],
                                    add=True)
            return carry

        lax.fori_loop(0, (_NCHUNK + _NB - 1) // _NB + 1, group, 0)

        plsc.subcore_barrier()
        pltpu.sync_copy(agg.at[pl.ds(row0, _RPT)],
                        out_hbm.at[c, pl.ds(row0, _RPT)])

        @pl.when(s == 0)
        def _():
            pltpu.sync_copy(agg.at[pl.ds(_RPT * _NS, _REM)],
                            out_hbm.at[c, pl.ds(_RPT * _NS, _REM)])

    return body(h, src, dst)


_BR = 1000
_NBLK = _N // _BR


def _tc_layer(h, p, eps, w1, b1, g, be, w2, b2, wf=None):
    final = wf is not None

    def kfn(h_ref, p_ref, eps_ref, w1_ref, b1_ref, g_ref, be_ref, w2_ref,
            b2_ref, *rest):
        if final:
            wf1_ref, bf1_ref, wf2_ref, bf2_ref = rest[:4]
            rest = rest[4:]
        o_ref, y_scr, acc_scr, stat_scr = rest
        i = pl.program_id(0)

        @pl.when(i < _NBLK)
        def _():
            eps_v = eps_ref[0]
            comb = (1.0 + eps_v) * h_ref[...] + p_ref[0] + p_ref[1]
            y = jnp.dot(comb, w1_ref[...],
                        preferred_element_type=jnp.float32) + b1_ref[...]
            y_scr[i] = y
            ss = jnp.concatenate([jnp.sum(y, axis=0, keepdims=True),
                                  jnp.sum(y * y, axis=0, keepdims=True)], 0)

            @pl.when(i == 0)
            def _():
                acc_scr[...] = ss

            @pl.when(i > 0)
            def _():
                acc_scr[...] = acc_scr[...] + ss

        @pl.when(i >= _NBLK)
        def _():
            @pl.when(i == _NBLK)
            def _():
                mu = acc_scr[0:1] / float(_N)
                var = acc_scr[1:2] / float(_N) - mu * mu
                stat_scr[...] = jnp.concatenate(
                    [mu, lax.rsqrt(var + 1e-5)], 0)

            y = y_scr[i - _NBLK]
            z = jnp.maximum((y - stat_scr[0:1]) * stat_scr[1:2] * g_ref[...]
                            + be_ref[...], 0.0)
            h2 = jnp.maximum(
                jnp.dot(z, w2_ref[...], preferred_element_type=jnp.float32)
                + b2_ref[...], 0.0)
            if final:
                t2 = jnp.maximum(
                    jnp.dot(h2, wf1_ref[...],
                            preferred_element_type=jnp.float32)
                    + bf1_ref[...], 0.0)
                h2 = (jnp.dot(t2, wf2_ref[...],
                              preferred_element_type=jnp.float32)
                      + bf2_ref[...])
            o_ref[...] = h2

    stay = lambda idx: (0, 0)
    mat = pl.BlockSpec((_D, _D), stay)
    vec = pl.BlockSpec((1, _D), stay)
    specs = [
        pl.BlockSpec((_BR, _D), lambda i: (jnp.minimum(i, _NBLK - 1), 0)),
        pl.BlockSpec((2, _BR, _D),
                     lambda i: (0, jnp.minimum(i, _NBLK - 1), 0)),
        pl.BlockSpec(memory_space=pltpu.SMEM),
        mat, vec, vec, vec, mat, vec,
    ]
    args = [h, p, eps.reshape(1), w1, b1.reshape(1, -1), g.reshape(1, -1),
            be.reshape(1, -1), w2, b2.reshape(1, -1)]
    if final:
        wf1, bf1, wf2, bf2 = wf
        specs += [mat, vec, mat, vec]
        args += [wf1, bf1.reshape(1, -1), wf2, bf2.reshape(1, -1)]
    return pl.pallas_call(
        kfn,
        grid=(2 * _NBLK,),
        out_shape=jax.ShapeDtypeStruct((_N, _D), jnp.float32),
        in_specs=specs,
        out_specs=pl.BlockSpec((_BR, _D),
                               lambda i: (jnp.maximum(i - _NBLK, 0), 0)),
        scratch_shapes=[
            pltpu.VMEM((_NBLK, _BR, _D), jnp.float32),
            pltpu.VMEM((2, _D), jnp.float32),
            pltpu.VMEM((2, _D), jnp.float32),
        ],
    )(*args)


def kernel(x, edge_index,
           eps0, W1_0, b1_0, g_0, be_0, W2_0, b2_0,
           eps1, W1_1, b1_1, g_1, be_1, W2_1, b2_1,
           eps2, W1_2, b1_2, g_2, be_2, W2_2, b2_2,
           Wf1, bf1, Wf2, bf2):
    src = edge_index[0]
    dst = edge_index[1]
    layers = [
        (eps0, W1_0, b1_0, g_0, be_0, W2_0, b2_0),
        (eps1, W1_1, b1_1, g_1, be_1, W2_1, b2_1),
        (eps2, W1_2, b1_2, g_2, be_2, W2_2, b2_2),
    ]
    h = x
    for l, (eps, w1, b1, g, be, w2, b2) in enumerate(layers):
        p = _seg_sum_sc(h, src, dst)
        if l < 2:
            h = _tc_layer(h, p, eps, w1, b1, g, be, w2, b2)
        else:
            h = _tc_layer(h, p, eps, w1, b1, g, be, w2, b2,
                          wf=(Wf1, bf1, Wf2, bf2))
    return h


# speedup vs baseline: 1.2391x; 1.0409x over previous
import functools

import jax
import jax.numpy as jnp
from jax import lax
from jax.experimental import pallas as pl
from jax.experimental.pallas import tpu as pltpu
from jax.experimental.pallas import tpu_sc as plsc

_N = 10000
_D = 128
_E = 320000
_NC = 2
_NS = 16
_NW = _NC * _NS
_EPW = _E // _NW
_C = 80
_NCHUNK = _EPW // _C
_NB = 4
_NI = 2 * _NB
_RPT = 624
_REM = _N - _RPT * _NS


def _seg_sum_sc(h, src, dst):
    mesh = plsc.VectorSubcoreMesh(core_axis_name="c", subcore_axis_name="s",
                                  num_cores=_NC, num_subcores=_NS)

    @functools.partial(
        pl.kernel,
        out_type=jax.ShapeDtypeStruct((_NC, _N, _D), jnp.float32),
        mesh=mesh,
        scratch_types=[
            pltpu.VMEM((_NB, _C), jnp.int32),
            pltpu.VMEM((_NB, _C), jnp.int32),
            pltpu.VMEM((_NB, _C, _D), jnp.float32),
            pltpu.VMEM_SHARED((_N, _D), jnp.float32),
            pltpu.SemaphoreType.DMA((_NB,)),
            pltpu.SemaphoreType.DMA((_NB,)),
        ],
    )
    def body(h_hbm, src_hbm, dst_hbm, out_hbm, idx_s, idx_d, rows, agg,
             sem_i, sem_g):
        c = lax.axis_index("c")
        s = lax.axis_index("s")
        wid = s * _NC + c
        base = wid * _EPW

        for b in range(_NB):
            off = base + b * _C
            pltpu.async_copy(src_hbm.at[pl.ds(off, _C)], idx_s.at[b],
                             sem_i.at[b])
            pltpu.async_copy(dst_hbm.at[pl.ds(off, _C)], idx_d.at[b],
                             sem_i.at[b])
        zb = rows.at[_NB - 1]

        def zstore(i, carry):
            r = i // (_D // 16)
            col = (i % (_D // 16)) * 16
            zb[r, pl.ds(col, 16)] = jnp.zeros((16,), jnp.float32)
            return carry

        lax.fori_loop(0, _C * (_D // 16), zstore, 0)
        for b in range(_NB - 1):
            off = base + b * _C
            pltpu.make_async_copy(src_hbm.at[pl.ds(off, _C)],
                                  idx_s.at[b], sem_i.at[b]).wait()
            pltpu.make_async_copy(dst_hbm.at[pl.ds(off, _C)],
                                  idx_d.at[b], sem_i.at[b]).wait()
            pltpu.async_copy(h_hbm.at[idx_s.at[b]], rows.at[b], sem_g.at[b])
        row0 = s * _RPT
        for k in range(_RPT // _C):
            pltpu.sync_copy(zb, agg.at[pl.ds(row0 + k * _C, _C)])
        pltpu.sync_copy(zb.at[pl.ds(0, _RPT % _C)],
                        agg.at[pl.ds(row0 + (_RPT // _C) * _C, _RPT % _C)])

        @pl.when(s == 0)
        def _():
            pltpu.sync_copy(zb.at[pl.ds(0, _REM)],
                            agg.at[pl.ds(_RPT * _NS, _REM)])

        plsc.subcore_barrier()

        def group(g, carry):
            for b in range(_NB):
                i = g * _NB + b
                ci, cg, cs = i, i - 1, i - 3

                @pl.when(jnp.logical_and(ci >= _NB, ci < _NCHUNK))
                def _():
                    off = base + ci * _C
                    pltpu.async_copy(src_hbm.at[pl.ds(off, _C)],
                                     idx_s.at[b], sem_i.at[b])
                    pltpu.async_copy(dst_hbm.at[pl.ds(off, _C)],
                                     idx_d.at[b], sem_i.at[b])

                bg = (b - 1) % _NB

                @pl.when(jnp.logical_and(cg >= _NB - 1, cg < _NCHUNK))
                def _():
                    off = base + cg * _C
                    pltpu.make_async_copy(src_hbm.at[pl.ds(off, _C)],
                                          idx_s.at[bg], sem_i.at[bg]).wait()
                    pltpu.make_async_copy(dst_hbm.at[pl.ds(off, _C)],
                                          idx_d.at[bg], sem_i.at[bg]).wait()
                    pltpu.async_copy(h_hbm.at[idx_s.at[bg]], rows.at[bg],
                                     sem_g.at[bg])

                bs = (b - 3) % _NB

                @pl.when(jnp.logical_and(cs >= 0, cs < _NCHUNK))
                def _():
                    pltpu.make_async_copy(h_hbm.at[idx_s.at[bs]],
                                          rows.at[bs], sem_g.at[bs]).wait()
                    pltpu.sync_copy(rows.at[bs], agg.at[idx_d.at[bs]],
                                    add=True)
            return carry

        lax.fori_loop(0, (_NCHUNK + _NB - 1) // _NB + 1, group, 0)

        plsc.subcore_barrier()
        pltpu.sync_copy(agg.at[pl.ds(row0, _RPT)],
                        out_hbm.at[c, pl.ds(row0, _RPT)])

        @pl.when(s == 0)
        def _():
            pltpu.sync_copy(agg.at[pl.ds(_RPT * _NS, _REM)],
                            out_hbm.at[c, pl.ds(_RPT * _NS, _REM)])

    return body(h, src, dst)


def _gin_mlp(h_ref, p_ref, eps_ref, w1_ref, b1_ref, g_ref, be_ref, w2_ref, b2_ref):
    eps = eps_ref[0]
    comb = (1.0 + eps) * h_ref[...] + p_ref[0] + p_ref[1]
    y = jnp.dot(comb, w1_ref[...], preferred_element_type=jnp.float32) + b1_ref[...]
    mu = jnp.mean(y, axis=0, keepdims=True)
    var = jnp.mean((y - mu) ** 2, axis=0, keepdims=True)
    yn = (y - mu) * lax.rsqrt(var + 1e-5) * g_ref[...] + be_ref[...]
    z = jnp.maximum(yn, 0.0)
    h2 = jnp.dot(z, w2_ref[...], preferred_element_type=jnp.float32) + b2_ref[...]
    return jnp.maximum(h2, 0.0)


def _tc_layer(h, p, eps, w1, b1, g, be, w2, b2):
    def kfn(h_ref, p_ref, eps_ref, w1_ref, b1_ref, g_ref, be_ref, w2_ref,
            b2_ref, o_ref):
        o_ref[...] = _gin_mlp(h_ref, p_ref, eps_ref, w1_ref, b1_ref, g_ref,
                              be_ref, w2_ref, b2_ref)

    specs = [pl.BlockSpec(memory_space=pltpu.VMEM) for _ in range(9)]
    specs[2] = pl.BlockSpec(memory_space=pltpu.SMEM)
    return pl.pallas_call(
        kfn,
        out_shape=jax.ShapeDtypeStruct((_N, _D), jnp.float32),
        in_specs=specs,
        out_specs=pl.BlockSpec(memory_space=pltpu.VMEM),
    )(h, p, eps.reshape(1), w1, b1.reshape(1, -1), g.reshape(1, -1),
      be.reshape(1, -1), w2, b2.reshape(1, -1))


def _tc_layer_final(h, p, eps, w1, b1, g, be, w2, b2, wf1, bf1, wf2, bf2):
    def kfn(h_ref, p_ref, eps_ref, w1_ref, b1_ref, g_ref, be_ref, w2_ref,
            b2_ref, wf1_ref, bf1_ref, wf2_ref, bf2_ref, o_ref):
        h2 = _gin_mlp(h_ref, p_ref, eps_ref, w1_ref, b1_ref, g_ref, be_ref,
                      w2_ref, b2_ref)
        t = jnp.maximum(
            jnp.dot(h2, wf1_ref[...], preferred_element_type=jnp.float32)
            + bf1_ref[...], 0.0)
        o_ref[...] = (jnp.dot(t, wf2_ref[...], preferred_element_type=jnp.float32)
                      + bf2_ref[...])

    specs = [pl.BlockSpec(memory_space=pltpu.VMEM) for _ in range(13)]
    specs[2] = pl.BlockSpec(memory_space=pltpu.SMEM)
    return pl.pallas_call(
        kfn,
        out_shape=jax.ShapeDtypeStruct((_N, _D), jnp.float32),
        in_specs=specs,
        out_specs=pl.BlockSpec(memory_space=pltpu.VMEM),
    )(h, p, eps.reshape(1), w1, b1.reshape(1, -1), g.reshape(1, -1),
      be.reshape(1, -1), w2, b2.reshape(1, -1), wf1, bf1.reshape(1, -1),
      wf2, bf2.reshape(1, -1))


def kernel(x, edge_index,
           eps0, W1_0, b1_0, g_0, be_0, W2_0, b2_0,
           eps1, W1_1, b1_1, g_1, be_1, W2_1, b2_1,
           eps2, W1_2, b1_2, g_2, be_2, W2_2, b2_2,
           Wf1, bf1, Wf2, bf2):
    src = edge_index[0]
    dst = edge_index[1]
    layers = [
        (eps0, W1_0, b1_0, g_0, be_0, W2_0, b2_0),
        (eps1, W1_1, b1_1, g_1, be_1, W2_1, b2_1),
        (eps2, W1_2, b1_2, g_2, be_2, W2_2, b2_2),
    ]
    h = x
    for l, (eps, w1, b1, g, be, w2, b2) in enumerate(layers):
        p = _seg_sum_sc(h, src, dst)
        if l < 2:
            h = _tc_layer(h, p, eps, w1, b1, g, be, w2, b2)
        else:
            h = _tc_layer_final(h, p, eps, w1, b1, g, be, w2, b2,
                                Wf1, bf1, Wf2, bf2)
    return h
